# Initial kernel scaffold; baseline (speedup 1.0000x reference)
import jax
import jax.numpy as jnp
from jax.experimental import pallas as pl


def kernel(t, y, edge_index, W1, b1, W2, b2, W3, b3):
    raise NotImplementedError("write your pallas kernel here")


# speedup vs baseline: 2.9259x; 2.9259x over previous
import jax
import jax.numpy as jnp
from jax import lax
from jax.experimental import pallas as pl
from jax.experimental.pallas import tpu as pltpu
from jax.experimental.pallas import tpu_sc as plsc

_N = 10000
_E = 320000
_D = 128
_CHUNK = 128
_NC = 2
_NS = 16
_NW = _NC * _NS
_CPW = 80
_NUM_CHUNKS = _CPW * _NW
_E_PAD = _NUM_CHUNKS * _CHUNK
_EPW = _CPW * _CHUNK
_N_PAD = 10240
_RPT = _N_PAD // _NS


def _spmm_body(z_hbm, src_hbm, dst_hbm, zeros_hbm, out_hbm,
               src_v, dst_v, rows_v, acc, sem):
    c = lax.axis_index("c")
    s = lax.axis_index("s")
    wid = s * _NC + c

    pltpu.sync_copy(zeros_hbm, acc.at[pl.ds(s * _RPT, _RPT)])

    pltpu.sync_copy(src_hbm.at[pl.ds(wid * _EPW, _EPW)], src_v)
    pltpu.sync_copy(dst_hbm.at[pl.ds(wid * _EPW, _EPW)], dst_v)

    plsc.subcore_barrier()

    @pl.loop(0, _CPW)
    def _(i):
        pltpu.async_copy(z_hbm.at[src_v.at[pl.ds(i * _CHUNK, _CHUNK)]],
                         rows_v, sem).wait()
        pltpu.sync_copy(rows_v, acc.at[dst_v.at[pl.ds(i * _CHUNK, _CHUNK)]],
                        add=True)

    plsc.subcore_barrier()
    pltpu.sync_copy(acc.at[pl.ds(s * _RPT, _RPT)],
                    out_hbm.at[c, pl.ds(s * _RPT, _RPT)])


_spmm = pl.kernel(
    _spmm_body,
    out_type=jax.ShapeDtypeStruct((_NC, _N_PAD, _D), jnp.float32),
    mesh=plsc.VectorSubcoreMesh(core_axis_name="c", subcore_axis_name="s",
                                num_cores=_NC, num_subcores=_NS),
    scratch_types=[
        pltpu.VMEM((_EPW,), jnp.int32),
        pltpu.VMEM((_EPW,), jnp.int32),
        pltpu.VMEM((_CHUNK, _D), jnp.float32),
        pltpu.VMEM_SHARED((_N_PAD, _D), jnp.float32),
        pltpu.SemaphoreType.DMA,
    ],
)


def _mm_first_body(y_ref, w_ref, b_ref, o_ref):
    o_ref[...] = lax.dot_general(
        y_ref[...], w_ref[...], (((1,), (1,)), ((), ())),
        preferred_element_type=jnp.float32) + b_ref[...]


_mm_first = pl.pallas_call(
    _mm_first_body,
    out_shape=jax.ShapeDtypeStruct((_N, _D), jnp.float32),
)


def _mm_mid_body(p_ref, w_ref, b_ref, o_ref):
    x = jnp.maximum(p_ref[0, :_N] + p_ref[1, :_N], 0.0)
    o_ref[...] = lax.dot_general(
        x, w_ref[...], (((1,), (1,)), ((), ())),
        preferred_element_type=jnp.float32) + b_ref[...]


_mm_mid = pl.pallas_call(
    _mm_mid_body,
    out_shape=jax.ShapeDtypeStruct((_N, _D), jnp.float32),
)


def _sum_body(p_ref, o_ref):
    o_ref[...] = p_ref[0, :_N] + p_ref[1, :_N]


_sum_partials = pl.pallas_call(
    _sum_body,
    out_shape=jax.ShapeDtypeStruct((_N, _D), jnp.float32),
)


def kernel(t, y, edge_index, W1, b1, W2, b2, W3, b3):
    pad_src = jnp.zeros((_E_PAD - _E,), jnp.int32)
    pad_dst = jnp.full((_E_PAD - _E,), _N, jnp.int32)
    src = jnp.concatenate([edge_index[0], pad_src])
    dst = jnp.concatenate([edge_index[1], pad_dst])
    zeros = jnp.zeros((_RPT, _D), jnp.float32)

    z = _mm_first(y, W1, b1.reshape(1, _D))
    p = _spmm(z, src, dst, zeros)
    z = _mm_mid(p, W2, b2.reshape(1, _D))
    p = _spmm(z, src, dst, zeros)
    z = _mm_mid(p, W3, b3.reshape(1, _D))
    p = _spmm(z, src, dst, zeros)
    return _sum_partials(p)


# speedup vs baseline: 3.4623x; 1.1833x over previous
import jax
import jax.numpy as jnp
from jax import lax
from jax.experimental import pallas as pl
from jax.experimental.pallas import tpu as pltpu
from jax.experimental.pallas import tpu_sc as plsc

_N = 10000
_E = 320000
_D = 128
_CHUNK = 128
_NC = 2
_NS = 16
_NW = _NC * _NS
_CPW = 80
_NUM_CHUNKS = _CPW * _NW
_E_PAD = _NUM_CHUNKS * _CHUNK
_EPW = _CPW * _CHUNK
_NACC = 10112
_RPT = _NACC // _NS
_NBUF = 2
_NGRP = _CPW // _NBUF


_EPH = _EPW // 2
_CPH = _CPW // 2


def _spmm_body(z_hbm, src_hbm, dst_hbm, zeros_hbm, out_hbm,
               src_v, dst_v, didx0, didx1, rows0, rows1, acc,
               gsem0, gsem1, ssem0, ssem1):
    rows = (rows0, rows1)
    didx = (didx0, didx1)
    gsem = (gsem0, gsem1)
    ssem = (ssem0, ssem1)
    c = lax.axis_index("c")
    s = lax.axis_index("s")
    wid = s * _NC + c

    def stage(h):
        pltpu.sync_copy(src_hbm.at[pl.ds(wid * _EPW + h * _EPH, _EPH)],
                        src_v)
        pltpu.sync_copy(dst_hbm.at[pl.ds(wid * _EPW + h * _EPH, _EPH)],
                        dst_v)

    pltpu.sync_copy(zeros_hbm, acc.at[pl.ds(s * _RPT, _RPT)])
    stage(0)
    plsc.subcore_barrier()

    def wait_scatter(b):
        pltpu.make_async_copy(rows[b], acc.at[didx[b]], ssem[b]).wait()

    def visit(lch, b):
        for k in range(8):
            didx[b][pl.ds(k * 16, 16)] = dst_v[pl.ds(lch * _CHUNK + k * 16,
                                                     16)]
        pltpu.async_copy(z_hbm.at[src_v.at[pl.ds(lch * _CHUNK, _CHUNK)]],
                         rows[b], gsem[b]).wait()
        pltpu.async_copy(rows[b], acc.at[didx[b]], ssem[b], add=True)

    visit(0, 0)
    visit(1, 1)

    @pl.loop(1, _CPH // 2)
    def _(g):
        for b in range(_NBUF):
            wait_scatter(b)
            visit(g * _NBUF + b, b)

    stage(1)

    @pl.loop(0, _CPH // 2)
    def _(g):
        for b in range(_NBUF):
            wait_scatter(b)
            visit(g * _NBUF + b, b)

    for b in range(_NBUF):
        wait_scatter(b)

    plsc.subcore_barrier()
    pltpu.sync_copy(acc.at[pl.ds(s * _RPT, _RPT)],
                    out_hbm.at[c, pl.ds(s * _RPT, _RPT)])


_spmm = pl.kernel(
    _spmm_body,
    out_type=jax.ShapeDtypeStruct((_NC, _NACC, _D), jnp.float32),
    mesh=plsc.VectorSubcoreMesh(core_axis_name="c", subcore_axis_name="s",
                                num_cores=_NC, num_subcores=_NS),
    scratch_types=[
        pltpu.VMEM((_EPH,), jnp.int32),
        pltpu.VMEM((_EPH,), jnp.int32),
        pltpu.VMEM((_CHUNK,), jnp.int32),
        pltpu.VMEM((_CHUNK,), jnp.int32),
        pltpu.VMEM((_CHUNK, _D), jnp.float32),
        pltpu.VMEM((_CHUNK, _D), jnp.float32),
        pltpu.VMEM_SHARED((_NACC, _D), jnp.float32),
    ] + [pltpu.SemaphoreType.DMA] * (2 * _NBUF),
)


def _mm_first_body(y_ref, w_ref, b_ref, o_ref):
    o_ref[...] = lax.dot_general(
        y_ref[...], w_ref[...], (((1,), (1,)), ((), ())),
        preferred_element_type=jnp.float32) + b_ref[...]


_mm_first = pl.pallas_call(
    _mm_first_body,
    out_shape=jax.ShapeDtypeStruct((_N, _D), jnp.float32),
)


def _mm_mid_body(p_ref, w_ref, b_ref, o_ref):
    x = jnp.maximum(p_ref[0, :_N] + p_ref[1, :_N], 0.0)
    o_ref[...] = lax.dot_general(
        x, w_ref[...], (((1,), (1,)), ((), ())),
        preferred_element_type=jnp.float32) + b_ref[...]


_mm_mid = pl.pallas_call(
    _mm_mid_body,
    out_shape=jax.ShapeDtypeStruct((_N, _D), jnp.float32),
)


def _sum_body(p_ref, o_ref):
    o_ref[...] = p_ref[0, :_N] + p_ref[1, :_N]


_sum_partials = pl.pallas_call(
    _sum_body,
    out_shape=jax.ShapeDtypeStruct((_N, _D), jnp.float32),
)


def kernel(t, y, edge_index, W1, b1, W2, b2, W3, b3):
    pad_src = jnp.zeros((_E_PAD - _E,), jnp.int32)
    pad_dst = _N + jnp.arange(_E_PAD - _E, dtype=jnp.int32) % (_NACC - _N)
    src = jnp.concatenate([edge_index[0], pad_src])
    dst = jnp.concatenate([edge_index[1], pad_dst])
    zeros = jnp.zeros((_RPT, _D), jnp.float32)

    z = _mm_first(y, W1, b1.reshape(1, _D))
    p = _spmm(z, src, dst, zeros)
    z = _mm_mid(p, W2, b2.reshape(1, _D))
    p = _spmm(z, src, dst, zeros)
    z = _mm_mid(p, W3, b3.reshape(1, _D))
    p = _spmm(z, src, dst, zeros)
    return _sum_partials(p)
